# baseline (speedup 1.0000x reference)
import functools

import jax
import jax.numpy as jnp
from jax import lax
from jax.experimental import pallas as pl
from jax.experimental.pallas import tpu as pltpu
from jax.experimental.pallas import tpu_sc as plsc

B, L, V, H, A = 4096, 50, 100001, 128, 8
NW = 32
BPW = B // NW
LANES = 16
NCH = H // LANES


def _bag_body(states, table, out, idx_flat, idx_v, buf0, buf1, acc, sem0, sem1):
    cid = lax.axis_index("c")
    sid = lax.axis_index("s")
    wid = sid * 2 + cid

    pltpu.sync_copy(states.at[pl.ds(wid * BPW * L, BPW * L)], idx_flat)

    row_l = [(lax.iota(jnp.int32, LANES) + 16 * g) * L for g in range(BPW // LANES)]

    def tbody(r, carry):
        for g in range(BPW // LANES):
            v = plsc.load_gather(idx_flat, [row_l[g] + r])
            idx_v[r, pl.ds(g * LANES, LANES)] = v
        return carry

    lax.fori_loop(0, L, tbody, 0, unroll=False)

    bufs = (buf0, buf1)
    sems = (sem0, sem1)

    def start(r, which):
        pltpu.make_async_copy(table.at[idx_v.at[r]], bufs[which], sems[which]).start()

    def wait(which):
        pltpu.make_async_copy(table.at[idx_v.at[0]], bufs[which], sems[which]).wait()

    def accum(buf, first):
        def jbody(j4, carry):
            for jj in range(4):
                j = j4 * 4 + jj
                for c in range(NCH):
                    sl = pl.ds(c * LANES, LANES)
                    v = buf[j, sl]
                    if first:
                        acc[j, sl] = v
                    else:
                        plsc.addupdate(acc.at[j, sl], v)
            return carry
        lax.fori_loop(0, BPW // 4, jbody, 0, unroll=False)

    start(0, 0)
    start(1, 1)

    wait(0)
    accum(buf0, first=True)
    start(2, 0)

    def pair(g, carry):
        r = 2 * g + 1
        wait(1)
        accum(buf1, first=False)

        @pl.when(r + 2 < L)
        def _():
            start(r + 2, 1)

        wait(0)
        accum(buf0, first=False)

        @pl.when(r + 3 < L)
        def _():
            start(r + 3, 0)

        return carry

    lax.fori_loop(0, (L - 2) // 2, pair, 0, unroll=False)

    wait(1)
    accum(buf1, first=False)

    pltpu.sync_copy(acc, out.at[pl.ds(wid * BPW, BPW)])


@jax.jit
def _bag_sum(states, table):
    mesh = plsc.VectorSubcoreMesh(core_axis_name="c", subcore_axis_name="s")
    k = functools.partial(
        pl.kernel,
        out_type=jax.ShapeDtypeStruct((B, H), jnp.float32),
        mesh=mesh,
        compiler_params=pltpu.CompilerParams(needs_layout_passes=False),
        scratch_types=[
            pltpu.VMEM((BPW * L,), jnp.int32),
            pltpu.VMEM((L, BPW), jnp.int32),
            pltpu.VMEM((BPW, H), jnp.float32),
            pltpu.VMEM((BPW, H), jnp.float32),
            pltpu.VMEM((BPW, H), jnp.float32),
            pltpu.SemaphoreType.DMA,
            pltpu.SemaphoreType.DMA,
        ],
    )(_bag_body)
    return k(states, table)


def _head_body(bag_ref, w_ref, b_ref, out_ref):
    x = jnp.maximum(bag_ref[...], 0.0)
    z = jnp.dot(x, w_ref[...], preferred_element_type=jnp.float32) + b_ref[...]
    mus = jnp.tanh(z[:, :A])
    sds = jax.nn.softplus(z[:, A:]) + 0.001
    out_ref[...] = jnp.concatenate([mus, sds], axis=1)


@jax.jit
def _heads(bag, wc, bc):
    blk = 512
    return pl.pallas_call(
        _head_body,
        grid=(B // blk,),
        in_specs=[
            pl.BlockSpec((blk, H), lambda i: (i, 0)),
            pl.BlockSpec((H, 2 * A), lambda i: (0, 0)),
            pl.BlockSpec((1, 2 * A), lambda i: (0, 0)),
        ],
        out_specs=pl.BlockSpec((blk, 2 * A), lambda i: (i, 0)),
        out_shape=jax.ShapeDtypeStruct((B, 2 * A), jnp.float32),
    )(bag, wc, bc)


def kernel(states, table, W_mu, b_mu, W_sd, b_sd):
    states_flat = states.astype(jnp.int32).reshape(-1)
    bag = _bag_sum(states_flat, table)
    wc = jnp.concatenate([W_mu, W_sd], axis=1)
    bc = jnp.concatenate([b_mu, b_sd])[None, :]
    out = _heads(bag, wc, bc)
    return out[:, :A], out[:, A:]


# speedup vs baseline: 1.0611x; 1.0611x over previous
import functools

import jax
import jax.numpy as jnp
from jax import lax
from jax.experimental import pallas as pl
from jax.experimental.pallas import tpu as pltpu
from jax.experimental.pallas import tpu_sc as plsc

B, L, V, H, A = 4096, 50, 100001, 128, 8
NW = 32
BPW = B // NW
LANES = 16
NCH = H // LANES
NBUF = 4


def _bag_body(states_t, table, out, idx_v, bufA, buf0, buf1, buf2, buf3,
              acc, semA, sem0, sem1, sem2, sem3):
    cid = lax.axis_index("c")
    sid = lax.axis_index("s")
    wid = sid * 2 + cid
    col0 = wid * BPW

    pltpu.sync_copy(states_t.at[:, pl.ds(col0, BPW)], idx_v)

    bufs = (buf0, buf1, buf2, buf3)
    sems = (sem0, sem1, sem2, sem3)

    def start(r, buf, sem):
        pltpu.make_async_copy(table.at[idx_v.at[r]], buf, sem).start()

    def wait(buf, sem):
        pltpu.make_async_copy(table.at[idx_v.at[0]], buf, sem).wait()

    def accum(buf, first):
        def jbody(j8, carry):
            for jj in range(8):
                j = j8 * 8 + jj
                for c in range(NCH):
                    sl = pl.ds(c * LANES, LANES)
                    v = buf[j, sl]
                    if first:
                        acc[j, sl] = v
                    else:
                        plsc.addupdate(acc.at[j, sl], v)
            return carry
        lax.fori_loop(0, BPW // 8, jbody, 0, unroll=False)

    start(0, bufA, semA)
    for b in range(NBUF):
        start(1 + b, bufs[b], sems[b])

    wait(bufA, semA)
    accum(bufA, first=True)

    def group(g, carry):
        for b in range(NBUF):
            r = NBUF * g + 1 + b
            wait(bufs[b], sems[b])
            accum(bufs[b], first=False)

            @pl.when(r + NBUF < L)
            def _():
                start(r + NBUF, bufs[b], sems[b])

        return carry

    lax.fori_loop(0, (L - 2) // NBUF, group, 0, unroll=False)

    wait(buf0, sem0)
    accum(buf0, first=False)

    pltpu.sync_copy(acc, out.at[pl.ds(wid * BPW, BPW)])


@jax.jit
def _bag_sum(states_t, table):
    mesh = plsc.VectorSubcoreMesh(core_axis_name="c", subcore_axis_name="s")
    k = functools.partial(
        pl.kernel,
        out_type=jax.ShapeDtypeStruct((B, H), jnp.float32),
        mesh=mesh,
        scratch_types=[
            pltpu.VMEM((L, BPW), jnp.int32),
            pltpu.VMEM((BPW, H), jnp.float32),
            pltpu.VMEM((BPW, H), jnp.float32),
            pltpu.VMEM((BPW, H), jnp.float32),
            pltpu.VMEM((BPW, H), jnp.float32),
            pltpu.VMEM((BPW, H), jnp.float32),
            pltpu.VMEM((BPW, H), jnp.float32),
            pltpu.SemaphoreType.DMA,
            pltpu.SemaphoreType.DMA,
            pltpu.SemaphoreType.DMA,
            pltpu.SemaphoreType.DMA,
            pltpu.SemaphoreType.DMA,
        ],
    )(_bag_body)
    return k(states_t, table)


def _head_body(bag_ref, wmu_ref, bmu_ref, wsd_ref, bsd_ref, mu_ref, sd_ref):
    x = jnp.maximum(bag_ref[...], 0.0)
    zmu = jnp.dot(x, wmu_ref[...], preferred_element_type=jnp.float32)
    zsd = jnp.dot(x, wsd_ref[...], preferred_element_type=jnp.float32)
    mu_ref[...] = jnp.tanh(zmu + bmu_ref[...])
    sd_ref[...] = jax.nn.softplus(zsd + bsd_ref[...]) + 0.001


@jax.jit
def _heads(bag, wmu, bmu, wsd, bsd):
    blk = 512
    return pl.pallas_call(
        _head_body,
        grid=(B // blk,),
        in_specs=[
            pl.BlockSpec((blk, H), lambda i: (i, 0)),
            pl.BlockSpec((H, A), lambda i: (0, 0)),
            pl.BlockSpec((1, A), lambda i: (0, 0)),
            pl.BlockSpec((H, A), lambda i: (0, 0)),
            pl.BlockSpec((1, A), lambda i: (0, 0)),
        ],
        out_specs=[
            pl.BlockSpec((blk, A), lambda i: (i, 0)),
            pl.BlockSpec((blk, A), lambda i: (i, 0)),
        ],
        out_shape=[
            jax.ShapeDtypeStruct((B, A), jnp.float32),
            jax.ShapeDtypeStruct((B, A), jnp.float32),
        ],
    )(bag, wmu, bmu, wsd, bsd)


def kernel(states, table, W_mu, b_mu, W_sd, b_sd):
    states_t = states.astype(jnp.int32).T
    bag = _bag_sum(states_t, table)
    mus, sds = _heads(bag, W_mu, b_mu[None, :], W_sd, b_sd[None, :])
    return mus, sds


# speedup vs baseline: 1.0963x; 1.0331x over previous
import functools

import jax
import jax.numpy as jnp
from jax import lax
from jax.experimental import pallas as pl
from jax.experimental.pallas import tpu as pltpu
from jax.experimental.pallas import tpu_sc as plsc

B, L, V, H, A = 4096, 50, 100001, 128, 8
NW = 32
BPW = B // NW
LANES = 16
NCH = H // LANES
NBUF = 4


def _bag_body(states_t, table, out, idx_v, bufA, buf0, buf1, buf2, buf3,
              acc, semA, sem0, sem1, sem2, sem3):
    cid = lax.axis_index("c")
    sid = lax.axis_index("s")
    wid = sid * 2 + cid
    col0 = wid * BPW

    pltpu.sync_copy(states_t.at[:, pl.ds(col0, BPW)], idx_v)

    bufs = (buf0, buf1, buf2, buf3)
    sems = (sem0, sem1, sem2, sem3)

    def start(r, buf, sem):
        pltpu.make_async_copy(table.at[idx_v.at[r]], buf, sem).start()

    def wait(buf, sem):
        pltpu.make_async_copy(table.at[idx_v.at[0]], buf, sem).wait()

    def accum(buf, first):
        def jbody(j8, carry):
            for jj in range(8):
                j = j8 * 8 + jj
                for c in range(NCH):
                    sl = pl.ds(c * LANES, LANES)
                    v = buf[j, sl]
                    if first:
                        acc[j, sl] = v
                    else:
                        plsc.addupdate(acc.at[j, sl], v)
            return carry
        lax.fori_loop(0, BPW // 8, jbody, 0, unroll=False)

    start(0, bufA, semA)
    for b in range(NBUF):
        start(1 + b, bufs[b], sems[b])

    wait(bufA, semA)
    accum(bufA, first=True)

    def group(g, carry):
        for b in range(NBUF):
            r = NBUF * g + 1 + b
            wait(bufs[b], sems[b])
            accum(bufs[b], first=False)

            @pl.when(r + NBUF < L)
            def _():
                start(r + NBUF, bufs[b], sems[b])

        return carry

    lax.fori_loop(0, (L - 2) // NBUF, group, 0, unroll=False)

    wait(buf0, sem0)
    accum(buf0, first=False)

    pltpu.sync_copy(acc, out.at[pl.ds(wid * BPW, BPW)])


@jax.jit
def _bag_sum(states_t, table):
    mesh = plsc.VectorSubcoreMesh(core_axis_name="c", subcore_axis_name="s")
    k = functools.partial(
        pl.kernel,
        out_type=jax.ShapeDtypeStruct((B, H), jnp.float32),
        mesh=mesh,
        scratch_types=[
            pltpu.VMEM((L, BPW), jnp.int32),
            pltpu.VMEM((BPW, H), jnp.float32),
            pltpu.VMEM((BPW, H), jnp.float32),
            pltpu.VMEM((BPW, H), jnp.float32),
            pltpu.VMEM((BPW, H), jnp.float32),
            pltpu.VMEM((BPW, H), jnp.float32),
            pltpu.VMEM((BPW, H), jnp.float32),
            pltpu.SemaphoreType.DMA,
            pltpu.SemaphoreType.DMA,
            pltpu.SemaphoreType.DMA,
            pltpu.SemaphoreType.DMA,
            pltpu.SemaphoreType.DMA,
        ],
    )(_bag_body)
    return k(states_t, table)


def _head_body(bag_ref, wmu_ref, bmu_ref, wsd_ref, bsd_ref, mu_ref, sd_ref):
    x = jnp.maximum(bag_ref[...], 0.0)
    zmu = jnp.dot(x, wmu_ref[...], preferred_element_type=jnp.float32)
    zsd = jnp.dot(x, wsd_ref[...], preferred_element_type=jnp.float32)
    mu_ref[...] = jnp.tanh(zmu + bmu_ref[...])
    sd_ref[...] = jax.nn.softplus(zsd + bsd_ref[...]) + 0.001


@jax.jit
def _heads(bag, wmu, bmu, wsd, bsd):
    blk = 2048
    return pl.pallas_call(
        _head_body,
        grid=(B // blk,),
        in_specs=[
            pl.BlockSpec((blk, H), lambda i: (i, 0)),
            pl.BlockSpec((H, A), lambda i: (0, 0)),
            pl.BlockSpec((1, A), lambda i: (0, 0)),
            pl.BlockSpec((H, A), lambda i: (0, 0)),
            pl.BlockSpec((1, A), lambda i: (0, 0)),
        ],
        out_specs=[
            pl.BlockSpec((blk, A), lambda i: (i, 0)),
            pl.BlockSpec((blk, A), lambda i: (i, 0)),
        ],
        out_shape=[
            jax.ShapeDtypeStruct((B, A), jnp.float32),
            jax.ShapeDtypeStruct((B, A), jnp.float32),
        ],
    )(bag, wmu, bmu, wsd, bsd)


def kernel(states, table, W_mu, b_mu, W_sd, b_sd):
    states_t = states.astype(jnp.int32).T
    bag = _bag_sum(states_t, table)
    mus, sds = _heads(bag, W_mu, b_mu[None, :], W_sd, b_sd[None, :])
    return mus, sds
